# baseline (speedup 1.0000x reference)
import functools

import jax
import jax.numpy as jnp
from jax import lax
from jax.experimental import pallas as pl
from jax.experimental.pallas import tpu as pltpu
from jax.experimental.pallas import tpu_sc as plsc

N = 10000
E = 320000
D = 128
NC, NS, L = 2, 16, 16
NW = NC * NS
EPT = E // NW
B = 80
NB = EPT // B
ZR = 80
ZC = 8

_mesh = plsc.VectorSubcoreMesh(
    core_axis_name="c", subcore_axis_name="s", num_cores=NC, num_subcores=NS)


H = 8


def _make_sc_pass(CH, hoff):
    HB = B * H // L
    out_type = (
        jax.ShapeDtypeStruct((NC * N, D), jnp.float32),
        jax.ShapeDtypeStruct((NC * N, H), jnp.float32),
    )
    scratch_types = [
        pltpu.VMEM((2, B), jnp.int32),
        pltpu.VMEM((2, B), jnp.int32),
        pltpu.VMEM((2, B), jnp.int32),
        pltpu.VMEM((2, B, H), jnp.float32),
        pltpu.VMEM((2, B, H), jnp.float32),
        pltpu.VMEM((2, B, H), jnp.float32),
        pltpu.VMEM((2, B, D), jnp.float32),
        pltpu.VMEM((2, B, D), jnp.float32),
        pltpu.VMEM_SHARED((N, D), jnp.float32),
        pltpu.VMEM_SHARED((N, H), jnp.float32),
    ] + [pltpu.SemaphoreType.DMA] * 14

    @functools.partial(
        pl.kernel, out_type=out_type, mesh=_mesh,
        scratch_types=scratch_types,
        compiler_params=pltpu.CompilerParams(
            needs_layout_passes=False, use_tc_tiling_on_sc=False))
    def body(srcp, dstp, as_t, ad_t, h_t, acc_out, den_out,
             sidx2, didx2, didxs2, asv2, adv2, wv2, hrows2, msgv2,
             acc_sh, den_sh, *sems):
        cid = lax.axis_index("c")
        sid = lax.axis_index("s")
        zv = jnp.zeros((L,), jnp.float32)
        iot = lax.iota(jnp.int32, L)
        bufs = [
            (sidx2.at[i], didx2.at[i], didxs2.at[i], asv2.at[i], adv2.at[i],
             wv2.at[i], hrows2.at[i], msgv2.at[i],
             sems[7 * i:7 * i + 3],
             sems[7 * i + 3:7 * i + 5],
             sems[7 * i + 5],
             sems[7 * i + 6])
            for i in range(2)
        ]
        hrows_z = hrows2.at[0]
        wv_z = wv2.at[0]

        def zh(i, _):
            hrows_z[i // (D // L), pl.ds((i % (D // L)) * L, L)] = zv
            return 0
        lax.fori_loop(0, B * (D // L), zh, 0)

        def zw(i, _):
            p = i * L + iot
            plsc.store_scatter(wv_z, [p // H, p % H], zv)
            return 0
        lax.fori_loop(0, HB, zw, 0)

        def zcp(k, _):
            c = sid * ZC + k

            @pl.when(c < N // ZR)
            def _():
                pltpu.sync_copy(hrows_z.at[pl.ds(0, ZR)],
                                acc_sh.at[pl.ds(c * ZR, ZR)])
                pltpu.sync_copy(wv_z.at[pl.ds(0, ZR)],
                                den_sh.at[pl.ds(c * ZR, ZR)])
            return 0
        lax.fori_loop(0, ZC, zcp, 0)

        ebase = (cid * NS + sid) * EPT

        def prime(x, blk):
            sidx, didx, _, asv, adv, _, hrows, _, gs, _, _, _ = bufs[x]
            off = ebase + blk * B
            pltpu.sync_copy(srcp.at[pl.ds(off, B)], sidx)
            pltpu.sync_copy(dstp.at[pl.ds(off, B)], didx)
            pltpu.async_copy(as_t.at[sidx], asv, gs[0])
            pltpu.async_copy(ad_t.at[didx], adv, gs[1])
            pltpu.async_copy(h_t.at[sidx], hrows, gs[2])

        def compute(x, steady, pf, pf_blk):
            sidx, didx, didxs, asv, adv, wv, hrows, msgv, gs, isems, dsem, \
                asem = bufs[x]
            pltpu.make_async_copy(as_t.at[sidx], asv, gs[0]).wait()
            pltpu.make_async_copy(ad_t.at[didx], adv, gs[1]).wait()
            pltpu.make_async_copy(h_t.at[sidx], hrows, gs[2]).wait()

            @pl.when(steady)
            def _():
                pltpu.make_async_copy(wv, den_sh.at[didxs], dsem).wait()
                pltpu.make_async_copy(msgv, acc_sh.at[didxs], asem).wait()

            @plsc.parallel_loop(0, HB, 1, unroll=4)
            def wb(v):
                p = v * L + iot
                rows = p // H
                cols = p % H
                e = (plsc.load_gather(asv, [rows, cols])
                     + plsc.load_gather(adv, [rows, cols]))
                e = jnp.where(e >= 0, e, 0.2 * e)
                w = jnp.exp(e)
                plsc.store_scatter(wv, [rows, cols], w)

            for i in range(B // L):
                didxs[pl.ds(i * L, L)] = didx[pl.ds(i * L, L)]
            off = ebase + pf_blk * B
            if pf is None:
                pltpu.sync_copy(wv, den_sh.at[didxs], add=True)
            else:
                pltpu.async_copy(wv, den_sh.at[didxs], dsem, add=True)

                @pl.when(pf)
                def _():
                    pltpu.async_copy(srcp.at[pl.ds(off, B)], sidx, isems[0])
                    pltpu.async_copy(dstp.at[pl.ds(off, B)], didx, isems[1])

            @plsc.parallel_loop(0, B, 1, unroll=8)
            def mb(b):
                rowspl = jnp.broadcast_to(b, (L,))
                for hl in range(D // CH):
                    wvec = plsc.load_gather(
                        wv, [rowspl, jnp.full((L,), hoff + hl, jnp.int32)])
                    for j in range(CH // L):
                        c0 = hl * CH + j * L
                        msgv[b, pl.ds(c0, L)] = hrows[b, pl.ds(c0, L)] * wvec

            if pf is None:
                pltpu.sync_copy(msgv, acc_sh.at[didxs], add=True)
            else:
                pltpu.async_copy(msgv, acc_sh.at[didxs], asem, add=True)

                @pl.when(pf)
                def _():
                    pltpu.make_async_copy(
                        srcp.at[pl.ds(off, B)], sidx, isems[0]).wait()
                    pltpu.make_async_copy(
                        dstp.at[pl.ds(off, B)], didx, isems[1]).wait()
                    pltpu.async_copy(as_t.at[sidx], asv, gs[0])
                    pltpu.async_copy(ad_t.at[didx], adv, gs[1])
                    pltpu.async_copy(h_t.at[sidx], hrows, gs[2])

        prime(0, 0)
        prime(1, 1)
        plsc.subcore_barrier()
        true_ = jnp.bool_(True)

        def pair(k2, _):
            compute(0, k2 >= 1, true_, 2 * k2 + 2)
            compute(1, k2 >= 1, k2 < NB // 2 - 1, 2 * k2 + 3)
            return 0
        lax.fori_loop(0, NB // 2, pair, 0)
        compute(0, true_, None, 0)
        _, _, didxs_b, _, _, wv_b, _, msgv_b, _, _, dsem_b, asem_b = bufs[1]
        pltpu.make_async_copy(wv_b, den_sh.at[didxs_b], dsem_b).wait()
        pltpu.make_async_copy(msgv_b, acc_sh.at[didxs_b], asem_b).wait()
        plsc.subcore_barrier()

        def drain(k, _):
            c = sid * ZC + k

            @pl.when(c < N // ZR)
            def _():
                pltpu.sync_copy(acc_sh.at[pl.ds(c * ZR, ZR)],
                                acc_out.at[pl.ds(cid * N + c * ZR, ZR)])
                pltpu.sync_copy(den_sh.at[pl.ds(c * ZR, ZR)],
                                den_out.at[pl.ds(cid * N + c * ZR, ZR)])
            return 0
        lax.fori_loop(0, ZC, drain, 0)

    return body


_sc_pass1 = _make_sc_pass(16, 0)
_sc_pass2 = tuple(_make_sc_pass(64, 2 * g) for g in range(4))

BN = 400


def _tc_prep1(x, W1, asf1, adf1):
    def body(x_ref, w_ref, as_ref, ad_ref, h_out, aso, ado):
        h = jnp.dot(x_ref[...], w_ref[...], preferred_element_type=jnp.float32, precision=lax.Precision.HIGHEST)
        h_out[...] = h
        c = lax.broadcasted_iota(jnp.int32, (128, 8), 0)
        hh = lax.broadcasted_iota(jnp.int32, (128, 8), 1)
        S = (c // 16 == hh).astype(jnp.float32)
        aso[...] = jnp.dot(h * as_ref[...], S, preferred_element_type=jnp.float32, precision=lax.Precision.HIGHEST)
        ado[...] = jnp.dot(h * ad_ref[...], S, preferred_element_type=jnp.float32, precision=lax.Precision.HIGHEST)

    return pl.pallas_call(
        body,
        grid=(N // BN,),
        in_specs=[
            pl.BlockSpec((BN, 128), lambda i: (i, 0)),
            pl.BlockSpec((128, 128), lambda i: (0, 0)),
            pl.BlockSpec((1, 128), lambda i: (0, 0)),
            pl.BlockSpec((1, 128), lambda i: (0, 0)),
        ],
        out_specs=(
            pl.BlockSpec((BN, 128), lambda i: (i, 0)),
            pl.BlockSpec((BN, 8), lambda i: (i, 0)),
            pl.BlockSpec((BN, 8), lambda i: (i, 0)),
        ),
        out_shape=(
            jax.ShapeDtypeStruct((N, 128), jnp.float32),
            jax.ShapeDtypeStruct((N, 8), jnp.float32),
            jax.ShapeDtypeStruct((N, 8), jnp.float32),
        ),
    )(x, W1, asf1, adf1)


def _tc_mid(acc0, acc1, den0, den1, b1r, W2, asf2, adf2):
    def body(a0, a1, d0, d1, b1_ref, w2, as_ref, ad_ref, h2o, aso, ado):
        den = d0[...] + d1[...] + 1e-16
        hh = lax.broadcasted_iota(jnp.int32, (8, 128), 0)
        cc = lax.broadcasted_iota(jnp.int32, (8, 128), 1)
        R8 = (hh == cc // 16).astype(jnp.float32)
        denrep = jnp.dot(den, R8, preferred_element_type=jnp.float32, precision=lax.Precision.HIGHEST)
        v = (a0[...] + a1[...]) / denrep + b1_ref[...]
        x1 = jnp.where(v > 0, v, jnp.exp(v) - 1.0)
        h2 = jnp.dot(x1, w2[...], preferred_element_type=jnp.float32, precision=lax.Precision.HIGHEST)
        h2o[...] = h2
        c2 = lax.broadcasted_iota(jnp.int32, (512, 8), 0)
        h2i = lax.broadcasted_iota(jnp.int32, (512, 8), 1)
        S2 = (c2 // 64 == h2i).astype(jnp.float32)
        aso[...] = jnp.dot(h2 * as_ref[...], S2, preferred_element_type=jnp.float32, precision=lax.Precision.HIGHEST)
        ado[...] = jnp.dot(h2 * ad_ref[...], S2, preferred_element_type=jnp.float32, precision=lax.Precision.HIGHEST)

    return pl.pallas_call(
        body,
        grid=(N // BN,),
        in_specs=[
            pl.BlockSpec((BN, 128), lambda i: (i, 0)),
            pl.BlockSpec((BN, 128), lambda i: (i, 0)),
            pl.BlockSpec((BN, 8), lambda i: (i, 0)),
            pl.BlockSpec((BN, 8), lambda i: (i, 0)),
            pl.BlockSpec((1, 128), lambda i: (0, 0)),
            pl.BlockSpec((128, 512), lambda i: (0, 0)),
            pl.BlockSpec((1, 512), lambda i: (0, 0)),
            pl.BlockSpec((1, 512), lambda i: (0, 0)),
        ],
        out_specs=(
            pl.BlockSpec((BN, 512), lambda i: (i, 0)),
            pl.BlockSpec((BN, 8), lambda i: (i, 0)),
            pl.BlockSpec((BN, 8), lambda i: (i, 0)),
        ),
        out_shape=(
            jax.ShapeDtypeStruct((N, 512), jnp.float32),
            jax.ShapeDtypeStruct((N, 8), jnp.float32),
            jax.ShapeDtypeStruct((N, 8), jnp.float32),
        ),
    )(acc0, acc1, den0, den1, b1r, W2, asf2, adf2)


def _tc_final(accs, dens, b2r):
    def body(*refs):
        a = refs[0:8]
        d = refs[8:16]
        b2_ref = refs[16]
        out = refs[17]
        hh = lax.broadcasted_iota(jnp.int32, (2, 128), 0)
        cc = lax.broadcasted_iota(jnp.int32, (2, 128), 1)
        R2 = (hh == cc // 64).astype(jnp.float32)
        cf = lax.broadcasted_iota(jnp.int32, (128, 64), 0)
        of = lax.broadcasted_iota(jnp.int32, (128, 64), 1)
        F = (cf % 64 == of).astype(jnp.float32)
        tot = jnp.zeros((BN, 64), jnp.float32)
        for g in range(4):
            den = d[2 * g][...] + d[2 * g + 1][...] + 1e-16
            denrep = jnp.dot(den, R2, preferred_element_type=jnp.float32, precision=lax.Precision.HIGHEST)
            v = (a[2 * g][...] + a[2 * g + 1][...]) / denrep
            tot = tot + jnp.dot(v, F, preferred_element_type=jnp.float32, precision=lax.Precision.HIGHEST)
        out[...] = tot * (1.0 / 8.0) + b2_ref[...]

    return pl.pallas_call(
        body,
        grid=(N // BN,),
        in_specs=(
            [pl.BlockSpec((BN, 128), lambda i: (i, 0)) for _ in range(8)]
            + [pl.BlockSpec((BN, 2), lambda i: (i, 0)) for _ in range(8)]
            + [pl.BlockSpec((1, 64), lambda i: (0, 0))]
        ),
        out_specs=pl.BlockSpec((BN, 64), lambda i: (i, 0)),
        out_shape=jax.ShapeDtypeStruct((N, 64), jnp.float32),
    )(*accs, *dens, b2r)


def kernel(x, edge_index, W1, a_src1, a_dst1, b1, W2, a_src2, a_dst2, b2):
    ei = edge_index.astype(jnp.int32)
    srcp = ei[0]
    dstp = ei[1]

    h1, as1, ad1 = _tc_prep1(x, W1, a_src1.reshape(1, -1), a_dst1.reshape(1, -1))
    accp, denp = _sc_pass1(srcp, dstp, as1, ad1, h1)
    h2, as2, ad2 = _tc_mid(accp[:N], accp[N:], denp[:N], denp[N:],
                           b1.reshape(1, -1), W2,
                           a_src2.reshape(1, -1), a_dst2.reshape(1, -1))
    accs, dens = [], []
    for g in range(4):
        a, dn = _sc_pass2[g](srcp, dstp, as2, ad2, h2[:, 128 * g:128 * (g + 1)])
        accs += [a[:N], a[N:]]
        dens += [dn[:N, 2 * g:2 * g + 2], dn[N:, 2 * g:2 * g + 2]]
    return _tc_final(accs, dens, b2.reshape(1, -1))


# speedup vs baseline: 1.1248x; 1.1248x over previous
import functools

import jax
import jax.numpy as jnp
from jax import lax
from jax.experimental import pallas as pl
from jax.experimental.pallas import tpu as pltpu
from jax.experimental.pallas import tpu_sc as plsc

N = 10000
E = 320000
D = 128
NC, NS, L = 2, 16, 16
NW = NC * NS
EPT = E // NW
B = 80
NB = EPT // B
ZR = 80
ZC = 8

_mesh = plsc.VectorSubcoreMesh(
    core_axis_name="c", subcore_axis_name="s", num_cores=NC, num_subcores=NS)


H = 8


def _make_sc_pass(CH, hoff):
    HB = B * H // L
    out_type = (
        jax.ShapeDtypeStruct((NC * N, D), jnp.float32),
        jax.ShapeDtypeStruct((NC * N, H), jnp.float32),
    )
    scratch_types = [
        pltpu.VMEM((2, B), jnp.int32),
        pltpu.VMEM((2, B), jnp.int32),
        pltpu.VMEM((2, B), jnp.int32),
        pltpu.VMEM((2, B, H), jnp.float32),
        pltpu.VMEM((2, B, H), jnp.float32),
        pltpu.VMEM((2, B, H), jnp.float32),
        pltpu.VMEM((2, B, D), jnp.float32),
        pltpu.VMEM((2, B, D), jnp.float32),
        pltpu.VMEM_SHARED((N, D), jnp.float32),
        pltpu.VMEM_SHARED((N, H), jnp.float32),
    ] + [pltpu.SemaphoreType.DMA] * 14

    @functools.partial(
        pl.kernel, out_type=out_type, mesh=_mesh,
        scratch_types=scratch_types,
        compiler_params=pltpu.CompilerParams(
            needs_layout_passes=False, use_tc_tiling_on_sc=False))
    def body(srcp, dstp, as_t, ad_t, h_t, acc_out, den_out,
             sidx2, didx2, didxs2, asv2, adv2, wv2, hrows2, msgv2,
             acc_sh, den_sh, *sems):
        cid = lax.axis_index("c")
        sid = lax.axis_index("s")
        zv = jnp.zeros((L,), jnp.float32)
        iot = lax.iota(jnp.int32, L)
        bufs = [
            (sidx2.at[i], didx2.at[i], didxs2.at[i], asv2.at[i], adv2.at[i],
             wv2.at[i], hrows2.at[i], msgv2.at[i],
             sems[7 * i:7 * i + 3],
             sems[7 * i + 3:7 * i + 5],
             sems[7 * i + 5],
             sems[7 * i + 6])
            for i in range(2)
        ]
        hrows_z = hrows2.at[0]
        wv_z = wv2.at[0]

        def zh(i, _):
            hrows_z[i // (D // L), pl.ds((i % (D // L)) * L, L)] = zv
            return 0
        lax.fori_loop(0, B * (D // L), zh, 0)

        def zw(i, _):
            p = i * L + iot
            plsc.store_scatter(wv_z, [p // H, p % H], zv)
            return 0
        lax.fori_loop(0, HB, zw, 0)

        def zcp(k, _):
            c = sid * ZC + k

            @pl.when(c < N // ZR)
            def _():
                pltpu.sync_copy(hrows_z.at[pl.ds(0, ZR)],
                                acc_sh.at[pl.ds(c * ZR, ZR)])
                pltpu.sync_copy(wv_z.at[pl.ds(0, ZR)],
                                den_sh.at[pl.ds(c * ZR, ZR)])
            return 0
        lax.fori_loop(0, ZC, zcp, 0)

        ebase = (cid * NS + sid) * EPT

        def prime(x, blk):
            sidx, didx, _, asv, adv, _, hrows, _, gs, _, _, _ = bufs[x]
            off = ebase + blk * B
            pltpu.sync_copy(srcp.at[pl.ds(off, B)], sidx)
            pltpu.sync_copy(dstp.at[pl.ds(off, B)], didx)
            pltpu.async_copy(as_t.at[sidx], asv, gs[0])
            pltpu.async_copy(ad_t.at[didx], adv, gs[1])
            pltpu.async_copy(h_t.at[sidx], hrows, gs[2])

        def compute(x, steady, pf, pf_blk):
            sidx, didx, didxs, asv, adv, wv, hrows, msgv, gs, isems, dsem, \
                asem = bufs[x]
            pltpu.make_async_copy(as_t.at[sidx], asv, gs[0]).wait()
            pltpu.make_async_copy(ad_t.at[didx], adv, gs[1]).wait()
            pltpu.make_async_copy(h_t.at[sidx], hrows, gs[2]).wait()

            @pl.when(steady)
            def _():
                pltpu.make_async_copy(wv, den_sh.at[didxs], dsem).wait()
                pltpu.make_async_copy(msgv, acc_sh.at[didxs], asem).wait()

            @plsc.parallel_loop(0, HB, 1, unroll=4)
            def wb(v):
                p = v * L + iot
                rows = p // H
                cols = p % H
                e = (plsc.load_gather(asv, [rows, cols])
                     + plsc.load_gather(adv, [rows, cols]))
                e = jnp.where(e >= 0, e, 0.2 * e)
                w = jnp.exp(e)
                plsc.store_scatter(wv, [rows, cols], w)

            for i in range(B // L):
                didxs[pl.ds(i * L, L)] = didx[pl.ds(i * L, L)]
            off = ebase + pf_blk * B
            if pf is None:
                pltpu.sync_copy(wv, den_sh.at[didxs], add=True)
            else:
                pltpu.async_copy(wv, den_sh.at[didxs], dsem, add=True)

                @pl.when(pf)
                def _():
                    pltpu.async_copy(srcp.at[pl.ds(off, B)], sidx, isems[0])
                    pltpu.async_copy(dstp.at[pl.ds(off, B)], didx, isems[1])

            @plsc.parallel_loop(0, B, 1, unroll=4)
            def mb(b):
                rowspl = jnp.broadcast_to(b, (L,))
                for hl in range(D // CH):
                    wvec = plsc.load_gather(
                        wv, [rowspl, jnp.full((L,), hoff + hl, jnp.int32)])
                    for j in range(CH // L):
                        c0 = hl * CH + j * L
                        msgv[b, pl.ds(c0, L)] = hrows[b, pl.ds(c0, L)] * wvec

            if pf is None:
                pltpu.sync_copy(msgv, acc_sh.at[didxs], add=True)
            else:
                pltpu.async_copy(msgv, acc_sh.at[didxs], asem, add=True)

                @pl.when(pf)
                def _():
                    pltpu.make_async_copy(
                        srcp.at[pl.ds(off, B)], sidx, isems[0]).wait()
                    pltpu.make_async_copy(
                        dstp.at[pl.ds(off, B)], didx, isems[1]).wait()
                    pltpu.async_copy(as_t.at[sidx], asv, gs[0])
                    pltpu.async_copy(ad_t.at[didx], adv, gs[1])
                    pltpu.async_copy(h_t.at[sidx], hrows, gs[2])

        prime(0, 0)
        prime(1, 1)
        plsc.subcore_barrier()
        true_ = jnp.bool_(True)

        def pair(k2, _):
            compute(0, k2 >= 1, true_, 2 * k2 + 2)
            compute(1, k2 >= 1, k2 < NB // 2 - 1, 2 * k2 + 3)
            return 0
        lax.fori_loop(0, NB // 2, pair, 0)
        compute(0, true_, None, 0)
        _, _, didxs_b, _, _, wv_b, _, msgv_b, _, _, dsem_b, asem_b = bufs[1]
        pltpu.make_async_copy(wv_b, den_sh.at[didxs_b], dsem_b).wait()
        pltpu.make_async_copy(msgv_b, acc_sh.at[didxs_b], asem_b).wait()
        plsc.subcore_barrier()

        def drain(k, _):
            c = sid * ZC + k

            @pl.when(c < N // ZR)
            def _():
                pltpu.sync_copy(acc_sh.at[pl.ds(c * ZR, ZR)],
                                acc_out.at[pl.ds(cid * N + c * ZR, ZR)])
                pltpu.sync_copy(den_sh.at[pl.ds(c * ZR, ZR)],
                                den_out.at[pl.ds(cid * N + c * ZR, ZR)])
            return 0
        lax.fori_loop(0, ZC, drain, 0)

    return body


_sc_pass1 = _make_sc_pass(16, 0)

EPT2 = E // NS
NB2 = EPT2 // B


def _make_sc_pass2m():
    CH = 64
    HB = B * H // L
    out_type = (
        jax.ShapeDtypeStruct((4 * N, D), jnp.float32),
        jax.ShapeDtypeStruct((NC * N, H), jnp.float32),
    )
    scratch_types = [
        pltpu.VMEM((2, B), jnp.int32),
        pltpu.VMEM((2, B), jnp.int32),
        pltpu.VMEM((2, B), jnp.int32),
        pltpu.VMEM((2, B), jnp.int32),
        pltpu.VMEM((2, B, H), jnp.float32),
        pltpu.VMEM((2, B, H), jnp.float32),
        pltpu.VMEM((2, B, H), jnp.float32),
        pltpu.VMEM((2, B, D), jnp.float32),
        pltpu.VMEM((2, B, D), jnp.float32),
        pltpu.VMEM_SHARED((N, D), jnp.float32),
        pltpu.VMEM_SHARED((N, H), jnp.float32),
    ] + [pltpu.SemaphoreType.DMA] * 14

    @functools.partial(
        pl.kernel, out_type=out_type, mesh=_mesh,
        scratch_types=scratch_types,
        compiler_params=pltpu.CompilerParams(
            needs_layout_passes=False, use_tc_tiling_on_sc=False))
    def body(srcp, dstp, as_t, ad_t, h_ts, acc_out, den_out,
             sidx2, didx2, didxs2, sidxh2, asv2, adv2, wv2, hrows2, msgv2,
             acc_sh, den_sh, *sems):
        cid = lax.axis_index("c")
        sid = lax.axis_index("s")
        zv = jnp.zeros((L,), jnp.float32)
        iot = lax.iota(jnp.int32, L)
        bufs = [
            (sidx2.at[i], didx2.at[i], didxs2.at[i], sidxh2.at[i],
             asv2.at[i], adv2.at[i], wv2.at[i], hrows2.at[i], msgv2.at[i],
             sems[7 * i:7 * i + 3], sems[7 * i + 3:7 * i + 5],
             sems[7 * i + 5], sems[7 * i + 6])
            for i in range(2)
        ]
        hrows_z = hrows2.at[0]
        wv_z = wv2.at[0]
        ebase = sid * EPT2

        def one_group(gl, _):
            g = 2 * cid + gl
            goff = g * N
            hoff = 2 * g

            def zh(i, _):
                hrows_z[i // (D // L), pl.ds((i % (D // L)) * L, L)] = zv
                return 0
            lax.fori_loop(0, B * (D // L), zh, 0)

            def zw(i, _):
                p = i * L + iot
                plsc.store_scatter(wv_z, [p // H, p % H], zv)
                return 0
            lax.fori_loop(0, HB, zw, 0)

            def zcp(k, _):
                c = sid * ZC + k

                @pl.when(c < N // ZR)
                def _():
                    pltpu.sync_copy(hrows_z.at[pl.ds(0, ZR)],
                                    acc_sh.at[pl.ds(c * ZR, ZR)])

                    @pl.when(gl == 0)
                    def _():
                        pltpu.sync_copy(wv_z.at[pl.ds(0, ZR)],
                                        den_sh.at[pl.ds(c * ZR, ZR)])
                return 0
            lax.fori_loop(0, ZC, zcp, 0)

            def prime(x, blk):
                sidx, didx, _, sidxh, asv, adv, _, hrows, _, gs, _, _, _ = \
                    bufs[x]
                off = ebase + blk * B
                pltpu.sync_copy(srcp.at[pl.ds(off, B)], sidx)
                pltpu.sync_copy(dstp.at[pl.ds(off, B)], didx)
                for i in range(B // L):
                    sidxh[pl.ds(i * L, L)] = sidx[pl.ds(i * L, L)] + goff
                pltpu.async_copy(as_t.at[sidx], asv, gs[0])
                pltpu.async_copy(ad_t.at[didx], adv, gs[1])
                pltpu.async_copy(h_ts.at[sidxh], hrows, gs[2])

            def compute(x, steady, pf, pf_blk):
                sidx, didx, didxs, sidxh, asv, adv, wv, hrows, msgv, gs, \
                    isems, dsem, asem = bufs[x]
                pltpu.make_async_copy(as_t.at[sidx], asv, gs[0]).wait()
                pltpu.make_async_copy(ad_t.at[didx], adv, gs[1]).wait()
                pltpu.make_async_copy(h_ts.at[sidxh], hrows, gs[2]).wait()

                @pl.when(steady)
                def _():
                    pltpu.make_async_copy(wv, den_sh.at[didxs], dsem).wait()
                    pltpu.make_async_copy(msgv, acc_sh.at[didxs], asem).wait()

                @plsc.parallel_loop(0, HB, 1, unroll=4)
                def wb(v):
                    p = v * L + iot
                    rows = p // H
                    cols = p % H
                    e = (plsc.load_gather(asv, [rows, cols])
                         + plsc.load_gather(adv, [rows, cols]))
                    e = jnp.where(e >= 0, e, 0.2 * e)
                    w = jnp.exp(e)
                    plsc.store_scatter(wv, [rows, cols], w)

                for i in range(B // L):
                    didxs[pl.ds(i * L, L)] = didx[pl.ds(i * L, L)]
                off = ebase + pf_blk * B
                pltpu.async_copy(wv, den_sh.at[didxs], dsem, add=True)

                @pl.when(pf)
                def _():
                    pltpu.async_copy(srcp.at[pl.ds(off, B)], sidx, isems[0])
                    pltpu.async_copy(dstp.at[pl.ds(off, B)], didx, isems[1])

                @plsc.parallel_loop(0, B, 1, unroll=4)
                def mb(b):
                    rowspl = jnp.broadcast_to(b, (L,))
                    for hl in range(D // CH):
                        wvec = plsc.load_gather(
                            wv, [rowspl, jnp.broadcast_to(hoff + hl, (L,))])
                        for j in range(CH // L):
                            c0 = hl * CH + j * L
                            msgv[b, pl.ds(c0, L)] = (
                                hrows[b, pl.ds(c0, L)] * wvec)

                pltpu.async_copy(msgv, acc_sh.at[didxs], asem, add=True)

                @pl.when(pf)
                def _():
                    pltpu.make_async_copy(
                        srcp.at[pl.ds(off, B)], sidx, isems[0]).wait()
                    pltpu.make_async_copy(
                        dstp.at[pl.ds(off, B)], didx, isems[1]).wait()
                    for i in range(B // L):
                        sidxh[pl.ds(i * L, L)] = sidx[pl.ds(i * L, L)] + goff
                    pltpu.async_copy(as_t.at[sidx], asv, gs[0])
                    pltpu.async_copy(ad_t.at[didx], adv, gs[1])
                    pltpu.async_copy(h_ts.at[sidxh], hrows, gs[2])

            prime(0, 0)
            prime(1, 1)
            plsc.subcore_barrier()

            def pair(k2, _):
                compute(0, k2 >= 1, k2 < NB2 // 2 - 1, 2 * k2 + 2)
                compute(1, k2 >= 1, k2 < NB2 // 2 - 1, 2 * k2 + 3)
                return 0
            lax.fori_loop(0, NB2 // 2, pair, 0)
            for x in range(2):
                _, _, didxs_x, _, _, _, wv_x, _, msgv_x, _, _, dsem_x, \
                    asem_x = bufs[x]
                pltpu.make_async_copy(wv_x, den_sh.at[didxs_x], dsem_x).wait()
                pltpu.make_async_copy(
                    msgv_x, acc_sh.at[didxs_x], asem_x).wait()
            plsc.subcore_barrier()

            def drain(k, _):
                c = sid * ZC + k

                @pl.when(c < N // ZR)
                def _():
                    pltpu.sync_copy(acc_sh.at[pl.ds(c * ZR, ZR)],
                                    acc_out.at[pl.ds(goff + c * ZR, ZR)])

                    @pl.when(gl == 1)
                    def _():
                        pltpu.sync_copy(
                            den_sh.at[pl.ds(c * ZR, ZR)],
                            den_out.at[pl.ds(cid * N + c * ZR, ZR)])
                return 0
            lax.fori_loop(0, ZC, drain, 0)
            plsc.subcore_barrier()
            return 0
        lax.fori_loop(0, 2, one_group, 0)

    return body


_sc_pass2m = _make_sc_pass2m()

BN = 400


def _tc_prep1(x, W1, asf1, adf1):
    def body(x_ref, w_ref, as_ref, ad_ref, h_out, aso, ado):
        h = jnp.dot(x_ref[...], w_ref[...], preferred_element_type=jnp.float32, precision=lax.Precision.HIGHEST)
        h_out[...] = h
        c = lax.broadcasted_iota(jnp.int32, (128, 8), 0)
        hh = lax.broadcasted_iota(jnp.int32, (128, 8), 1)
        S = (c // 16 == hh).astype(jnp.float32)
        aso[...] = jnp.dot(h * as_ref[...], S, preferred_element_type=jnp.float32, precision=lax.Precision.HIGHEST)
        ado[...] = jnp.dot(h * ad_ref[...], S, preferred_element_type=jnp.float32, precision=lax.Precision.HIGHEST)

    return pl.pallas_call(
        body,
        grid=(N // BN,),
        in_specs=[
            pl.BlockSpec((BN, 128), lambda i: (i, 0)),
            pl.BlockSpec((128, 128), lambda i: (0, 0)),
            pl.BlockSpec((1, 128), lambda i: (0, 0)),
            pl.BlockSpec((1, 128), lambda i: (0, 0)),
        ],
        out_specs=(
            pl.BlockSpec((BN, 128), lambda i: (i, 0)),
            pl.BlockSpec((BN, 8), lambda i: (i, 0)),
            pl.BlockSpec((BN, 8), lambda i: (i, 0)),
        ),
        out_shape=(
            jax.ShapeDtypeStruct((N, 128), jnp.float32),
            jax.ShapeDtypeStruct((N, 8), jnp.float32),
            jax.ShapeDtypeStruct((N, 8), jnp.float32),
        ),
    )(x, W1, asf1, adf1)


def _tc_mid(acc0, acc1, den0, den1, b1r, W2, asf2, adf2):
    def body(a0, a1, d0, d1, b1_ref, w2, as_ref, ad_ref, h2o, aso, ado):
        den = d0[...] + d1[...] + 1e-16
        hh = lax.broadcasted_iota(jnp.int32, (8, 128), 0)
        cc = lax.broadcasted_iota(jnp.int32, (8, 128), 1)
        R8 = (hh == cc // 16).astype(jnp.float32)
        denrep = jnp.dot(den, R8, preferred_element_type=jnp.float32, precision=lax.Precision.HIGHEST)
        v = (a0[...] + a1[...]) / denrep + b1_ref[...]
        x1 = jnp.where(v > 0, v, jnp.exp(v) - 1.0)
        h2 = jnp.dot(x1, w2[...], preferred_element_type=jnp.float32, precision=lax.Precision.HIGHEST)
        h2o[...] = h2
        c2 = lax.broadcasted_iota(jnp.int32, (512, 8), 0)
        h2i = lax.broadcasted_iota(jnp.int32, (512, 8), 1)
        S2 = (c2 // 64 == h2i).astype(jnp.float32)
        aso[...] = jnp.dot(h2 * as_ref[...], S2, preferred_element_type=jnp.float32, precision=lax.Precision.HIGHEST)
        ado[...] = jnp.dot(h2 * ad_ref[...], S2, preferred_element_type=jnp.float32, precision=lax.Precision.HIGHEST)

    return pl.pallas_call(
        body,
        grid=(N // BN,),
        in_specs=[
            pl.BlockSpec((BN, 128), lambda i: (i, 0)),
            pl.BlockSpec((BN, 128), lambda i: (i, 0)),
            pl.BlockSpec((BN, 8), lambda i: (i, 0)),
            pl.BlockSpec((BN, 8), lambda i: (i, 0)),
            pl.BlockSpec((1, 128), lambda i: (0, 0)),
            pl.BlockSpec((128, 512), lambda i: (0, 0)),
            pl.BlockSpec((1, 512), lambda i: (0, 0)),
            pl.BlockSpec((1, 512), lambda i: (0, 0)),
        ],
        out_specs=(
            pl.BlockSpec((BN, 512), lambda i: (i, 0)),
            pl.BlockSpec((BN, 8), lambda i: (i, 0)),
            pl.BlockSpec((BN, 8), lambda i: (i, 0)),
        ),
        out_shape=(
            jax.ShapeDtypeStruct((N, 512), jnp.float32),
            jax.ShapeDtypeStruct((N, 8), jnp.float32),
            jax.ShapeDtypeStruct((N, 8), jnp.float32),
        ),
    )(acc0, acc1, den0, den1, b1r, W2, asf2, adf2)


def _tc_final(accs, dens, b2r):
    def body(*refs):
        a = refs[0:4]
        d = refs[4:8]
        b2_ref = refs[8]
        out = refs[9]
        hh = lax.broadcasted_iota(jnp.int32, (2, 128), 0)
        cc = lax.broadcasted_iota(jnp.int32, (2, 128), 1)
        R2 = (hh == cc // 64).astype(jnp.float32)
        cf = lax.broadcasted_iota(jnp.int32, (128, 64), 0)
        of = lax.broadcasted_iota(jnp.int32, (128, 64), 1)
        F = (cf % 64 == of).astype(jnp.float32)
        tot = jnp.zeros((BN, 64), jnp.float32)
        for g in range(4):
            den = d[g][...] * 0.5 + 1e-16
            denrep = jnp.dot(den, R2, preferred_element_type=jnp.float32, precision=lax.Precision.HIGHEST)
            v = a[g][...] / denrep
            tot = tot + jnp.dot(v, F, preferred_element_type=jnp.float32, precision=lax.Precision.HIGHEST)
        out[...] = tot * (1.0 / 8.0) + b2_ref[...]

    return pl.pallas_call(
        body,
        grid=(N // BN,),
        in_specs=(
            [pl.BlockSpec((BN, 128), lambda i: (i, 0)) for _ in range(4)]
            + [pl.BlockSpec((BN, 2), lambda i: (i, 0)) for _ in range(4)]
            + [pl.BlockSpec((1, 64), lambda i: (0, 0))]
        ),
        out_specs=pl.BlockSpec((BN, 64), lambda i: (i, 0)),
        out_shape=jax.ShapeDtypeStruct((N, 64), jnp.float32),
    )(*accs, *dens, b2r)


def kernel(x, edge_index, W1, a_src1, a_dst1, b1, W2, a_src2, a_dst2, b2):
    ei = edge_index.astype(jnp.int32)
    srcp = ei[0]
    dstp = ei[1]

    h1, as1, ad1 = _tc_prep1(x, W1, a_src1.reshape(1, -1), a_dst1.reshape(1, -1))
    accp, denp = _sc_pass1(srcp, dstp, as1, ad1, h1)
    h2, as2, ad2 = _tc_mid(accp[:N], accp[N:], denp[:N], denp[N:],
                           b1.reshape(1, -1), W2,
                           a_src2.reshape(1, -1), a_dst2.reshape(1, -1))
    h2s = h2.reshape(N, 4, 128).transpose(1, 0, 2).reshape(4 * N, 128)
    acc4, den2o = _sc_pass2m(srcp, dstp, as2, ad2, h2s)
    accs = [acc4[g * N:(g + 1) * N] for g in range(4)]
    dens = [den2o[:N, 2 * g:2 * g + 2] for g in range(4)]
    return _tc_final(accs, dens, b2.reshape(1, -1))


# speedup vs baseline: 1.1534x; 1.0254x over previous
import functools

import jax
import jax.numpy as jnp
from jax import lax
from jax.experimental import pallas as pl
from jax.experimental.pallas import tpu as pltpu
from jax.experimental.pallas import tpu_sc as plsc

N = 10000
E = 320000
D = 128
NC, NS, L = 2, 16, 16
NW = NC * NS
EPT = E // NW
B = 80
NB = EPT // B
ZR = 80
ZC = 8

_mesh = plsc.VectorSubcoreMesh(
    core_axis_name="c", subcore_axis_name="s", num_cores=NC, num_subcores=NS)


H = 8


def _make_sc_pass(CH, hoff):
    HB = B * H // L
    out_type = (
        jax.ShapeDtypeStruct((NC * N, D), jnp.float32),
        jax.ShapeDtypeStruct((NC * N, H), jnp.float32),
    )
    scratch_types = [
        pltpu.VMEM((2, B), jnp.int32),
        pltpu.VMEM((2, B), jnp.int32),
        pltpu.VMEM((2, B), jnp.int32),
        pltpu.VMEM((2, B, H), jnp.float32),
        pltpu.VMEM((2, B, H), jnp.float32),
        pltpu.VMEM((2, B, H), jnp.float32),
        pltpu.VMEM((2, B, D), jnp.float32),
        pltpu.VMEM((2, B, D), jnp.float32),
        pltpu.VMEM_SHARED((N, D), jnp.float32),
        pltpu.VMEM_SHARED((N, H), jnp.float32),
    ] + [pltpu.SemaphoreType.DMA] * 14

    @functools.partial(
        pl.kernel, out_type=out_type, mesh=_mesh,
        scratch_types=scratch_types,
        compiler_params=pltpu.CompilerParams(
            needs_layout_passes=False, use_tc_tiling_on_sc=False))
    def body(srcp, dstp, as_t, ad_t, h_t, acc_out, den_out,
             sidx2, didx2, didxs2, asv2, adv2, wv2, hrows2, msgv2,
             acc_sh, den_sh, *sems):
        cid = lax.axis_index("c")
        sid = lax.axis_index("s")
        zv = jnp.zeros((L,), jnp.float32)
        iot = lax.iota(jnp.int32, L)
        bufs = [
            (sidx2.at[i], didx2.at[i], didxs2.at[i], asv2.at[i], adv2.at[i],
             wv2.at[i], hrows2.at[i], msgv2.at[i],
             sems[7 * i:7 * i + 3],
             sems[7 * i + 3:7 * i + 5],
             sems[7 * i + 5],
             sems[7 * i + 6])
            for i in range(2)
        ]
        hrows_z = hrows2.at[0]
        wv_z = wv2.at[0]

        def zh(i, _):
            hrows_z[i // (D // L), pl.ds((i % (D // L)) * L, L)] = zv
            return 0
        lax.fori_loop(0, B * (D // L), zh, 0)

        def zw(i, _):
            p = i * L + iot
            plsc.store_scatter(wv_z, [p // H, p % H], zv)
            return 0
        lax.fori_loop(0, HB, zw, 0)

        def zcp(k, _):
            c = sid * ZC + k

            @pl.when(c < N // ZR)
            def _():
                pltpu.sync_copy(hrows_z.at[pl.ds(0, ZR)],
                                acc_sh.at[pl.ds(c * ZR, ZR)])
                pltpu.sync_copy(wv_z.at[pl.ds(0, ZR)],
                                den_sh.at[pl.ds(c * ZR, ZR)])
            return 0
        lax.fori_loop(0, ZC, zcp, 0)

        ebase = (cid * NS + sid) * EPT

        def prime(x, blk):
            sidx, didx, _, asv, adv, _, hrows, _, gs, _, _, _ = bufs[x]
            off = ebase + blk * B
            pltpu.sync_copy(srcp.at[pl.ds(off, B)], sidx)
            pltpu.sync_copy(dstp.at[pl.ds(off, B)], didx)
            pltpu.async_copy(as_t.at[sidx], asv, gs[0])
            pltpu.async_copy(ad_t.at[didx], adv, gs[1])
            pltpu.async_copy(h_t.at[sidx], hrows, gs[2])

        def compute(x, steady, pf, pf_blk):
            sidx, didx, didxs, asv, adv, wv, hrows, msgv, gs, isems, dsem, \
                asem = bufs[x]
            pltpu.make_async_copy(as_t.at[sidx], asv, gs[0]).wait()
            pltpu.make_async_copy(ad_t.at[didx], adv, gs[1]).wait()
            pltpu.make_async_copy(h_t.at[sidx], hrows, gs[2]).wait()

            @pl.when(steady)
            def _():
                pltpu.make_async_copy(wv, den_sh.at[didxs], dsem).wait()
                pltpu.make_async_copy(msgv, acc_sh.at[didxs], asem).wait()

            @plsc.parallel_loop(0, HB, 1, unroll=4)
            def wb(v):
                p = v * L + iot
                rows = p // H
                cols = p % H
                e = (plsc.load_gather(asv, [rows, cols])
                     + plsc.load_gather(adv, [rows, cols]))
                e = jnp.where(e >= 0, e, 0.2 * e)
                w = jnp.exp(e)
                plsc.store_scatter(wv, [rows, cols], w)

            for i in range(B // L):
                didxs[pl.ds(i * L, L)] = didx[pl.ds(i * L, L)]
            off = ebase + pf_blk * B
            if pf is None:
                pltpu.sync_copy(wv, den_sh.at[didxs], add=True)
            else:
                pltpu.async_copy(wv, den_sh.at[didxs], dsem, add=True)

                @pl.when(pf)
                def _():
                    pltpu.async_copy(srcp.at[pl.ds(off, B)], sidx, isems[0])
                    pltpu.async_copy(dstp.at[pl.ds(off, B)], didx, isems[1])

            @plsc.parallel_loop(0, B, 1, unroll=4)
            def mb(b):
                rowspl = jnp.broadcast_to(b, (L,))
                for hl in range(D // CH):
                    wvec = plsc.load_gather(
                        wv, [rowspl, jnp.full((L,), hoff + hl, jnp.int32)])
                    for j in range(CH // L):
                        c0 = hl * CH + j * L
                        msgv[b, pl.ds(c0, L)] = hrows[b, pl.ds(c0, L)] * wvec

            if pf is None:
                pltpu.sync_copy(msgv, acc_sh.at[didxs], add=True)
            else:
                pltpu.async_copy(msgv, acc_sh.at[didxs], asem, add=True)

                @pl.when(pf)
                def _():
                    pltpu.make_async_copy(
                        srcp.at[pl.ds(off, B)], sidx, isems[0]).wait()
                    pltpu.make_async_copy(
                        dstp.at[pl.ds(off, B)], didx, isems[1]).wait()
                    pltpu.async_copy(as_t.at[sidx], asv, gs[0])
                    pltpu.async_copy(ad_t.at[didx], adv, gs[1])
                    pltpu.async_copy(h_t.at[sidx], hrows, gs[2])

        prime(0, 0)
        prime(1, 1)
        plsc.subcore_barrier()
        true_ = jnp.bool_(True)

        def pair(k2, _):
            compute(0, k2 >= 1, true_, 2 * k2 + 2)
            compute(1, k2 >= 1, k2 < NB // 2 - 1, 2 * k2 + 3)
            return 0
        lax.fori_loop(0, NB // 2, pair, 0)
        compute(0, true_, None, 0)
        _, _, didxs_b, _, _, wv_b, _, msgv_b, _, _, dsem_b, asem_b = bufs[1]
        pltpu.make_async_copy(wv_b, den_sh.at[didxs_b], dsem_b).wait()
        pltpu.make_async_copy(msgv_b, acc_sh.at[didxs_b], asem_b).wait()
        plsc.subcore_barrier()

        def drain(k, _):
            c = sid * ZC + k

            @pl.when(c < N // ZR)
            def _():
                pltpu.sync_copy(acc_sh.at[pl.ds(c * ZR, ZR)],
                                acc_out.at[pl.ds(cid * N + c * ZR, ZR)])
                pltpu.sync_copy(den_sh.at[pl.ds(c * ZR, ZR)],
                                den_out.at[pl.ds(cid * N + c * ZR, ZR)])
            return 0
        lax.fori_loop(0, ZC, drain, 0)

    return body


_sc_pass1 = _make_sc_pass(16, 0)

EPT2 = E // NS
NB2 = EPT2 // B


def _make_sc_pass2m():
    CH = 64
    HB = B * H // L
    out_type = (
        jax.ShapeDtypeStruct((4 * N, D), jnp.float32),
        jax.ShapeDtypeStruct((NC * N, H), jnp.float32),
    )
    scratch_types = [
        pltpu.VMEM((2, B), jnp.int32),
        pltpu.VMEM((2, B), jnp.int32),
        pltpu.VMEM((2, B), jnp.int32),
        pltpu.VMEM((2, B), jnp.int32),
        pltpu.VMEM((2, B, H), jnp.float32),
        pltpu.VMEM((2, B, H), jnp.float32),
        pltpu.VMEM((2, B, H), jnp.float32),
        pltpu.VMEM((2, B, D), jnp.bfloat16),
        pltpu.VMEM((2, B, D), jnp.float32),
        pltpu.VMEM_SHARED((N, D), jnp.float32),
        pltpu.VMEM_SHARED((N, H), jnp.float32),
    ] + [pltpu.SemaphoreType.DMA] * 14

    @functools.partial(
        pl.kernel, out_type=out_type, mesh=_mesh,
        scratch_types=scratch_types,
        compiler_params=pltpu.CompilerParams(
            needs_layout_passes=False, use_tc_tiling_on_sc=False))
    def body(srcp, dstp, as_t, ad_t, h_ts, acc_out, den_out,
             sidx2, didx2, didxs2, sidxh2, asv2, adv2, wv2, hrows2, msgv2,
             acc_sh, den_sh, *sems):
        cid = lax.axis_index("c")
        sid = lax.axis_index("s")
        zv = jnp.zeros((L,), jnp.float32)
        iot = lax.iota(jnp.int32, L)
        bufs = [
            (sidx2.at[i], didx2.at[i], didxs2.at[i], sidxh2.at[i],
             asv2.at[i], adv2.at[i], wv2.at[i], hrows2.at[i], msgv2.at[i],
             sems[7 * i:7 * i + 3], sems[7 * i + 3:7 * i + 5],
             sems[7 * i + 5], sems[7 * i + 6])
            for i in range(2)
        ]
        msgv_z = msgv2.at[0]
        wv_z = wv2.at[0]
        ebase = sid * EPT2

        def one_group(gl, _):
            g = 2 * cid + gl
            goff = g * N
            hoff = 2 * g

            def zh(i, _):
                msgv_z[i // (D // L), pl.ds((i % (D // L)) * L, L)] = zv
                return 0
            lax.fori_loop(0, B * (D // L), zh, 0)

            def zw(i, _):
                p = i * L + iot
                plsc.store_scatter(wv_z, [p // H, p % H], zv)
                return 0
            lax.fori_loop(0, HB, zw, 0)

            def zcp(k, _):
                c = sid * ZC + k

                @pl.when(c < N // ZR)
                def _():
                    pltpu.sync_copy(msgv_z.at[pl.ds(0, ZR)],
                                    acc_sh.at[pl.ds(c * ZR, ZR)])

                    @pl.when(gl == 0)
                    def _():
                        pltpu.sync_copy(wv_z.at[pl.ds(0, ZR)],
                                        den_sh.at[pl.ds(c * ZR, ZR)])
                return 0
            lax.fori_loop(0, ZC, zcp, 0)

            def prime(x, blk):
                sidx, didx, _, sidxh, asv, adv, _, hrows, _, gs, _, _, _ = \
                    bufs[x]
                off = ebase + blk * B
                pltpu.sync_copy(srcp.at[pl.ds(off, B)], sidx)
                pltpu.sync_copy(dstp.at[pl.ds(off, B)], didx)
                for i in range(B // L):
                    sidxh[pl.ds(i * L, L)] = sidx[pl.ds(i * L, L)] + goff
                pltpu.async_copy(as_t.at[sidx], asv, gs[0])
                pltpu.async_copy(ad_t.at[didx], adv, gs[1])
                pltpu.async_copy(h_ts.at[sidxh], hrows, gs[2])

            def compute(x, steady, pf, pf_blk):
                sidx, didx, didxs, sidxh, asv, adv, wv, hrows, msgv, gs, \
                    isems, dsem, asem = bufs[x]
                pltpu.make_async_copy(as_t.at[sidx], asv, gs[0]).wait()
                pltpu.make_async_copy(ad_t.at[didx], adv, gs[1]).wait()
                pltpu.make_async_copy(h_ts.at[sidxh], hrows, gs[2]).wait()

                @pl.when(steady)
                def _():
                    pltpu.make_async_copy(wv, den_sh.at[didxs], dsem).wait()
                    pltpu.make_async_copy(msgv, acc_sh.at[didxs], asem).wait()

                @plsc.parallel_loop(0, HB, 1, unroll=4)
                def wb(v):
                    p = v * L + iot
                    rows = p // H
                    cols = p % H
                    e = (plsc.load_gather(asv, [rows, cols])
                         + plsc.load_gather(adv, [rows, cols]))
                    e = jnp.where(e >= 0, e, 0.2 * e)
                    w = jnp.exp(e)
                    plsc.store_scatter(wv, [rows, cols], w)

                for i in range(B // L):
                    didxs[pl.ds(i * L, L)] = didx[pl.ds(i * L, L)]
                off = ebase + pf_blk * B
                pltpu.async_copy(wv, den_sh.at[didxs], dsem, add=True)

                @pl.when(pf)
                def _():
                    pltpu.async_copy(srcp.at[pl.ds(off, B)], sidx, isems[0])
                    pltpu.async_copy(dstp.at[pl.ds(off, B)], didx, isems[1])

                @plsc.parallel_loop(0, B, 1, unroll=4)
                def mb(b):
                    rowspl = jnp.broadcast_to(b, (L,))
                    for hl in range(D // CH):
                        wvec = plsc.load_gather(
                            wv, [rowspl, jnp.broadcast_to(hoff + hl, (L,))])
                        for j2 in range(CH // (2 * L)):
                            c32 = hl * CH + j2 * 2 * L
                            v = hrows[b, pl.ds(c32, 2 * L)]
                            lo, hi = plsc.unpack(
                                v, format=plsc.PackFormat.INTERLEAVED)
                            msgv[b, pl.ds(c32, L)] = lo * wvec
                            msgv[b, pl.ds(c32 + L, L)] = hi * wvec

                pltpu.async_copy(msgv, acc_sh.at[didxs], asem, add=True)

                @pl.when(pf)
                def _():
                    pltpu.make_async_copy(
                        srcp.at[pl.ds(off, B)], sidx, isems[0]).wait()
                    pltpu.make_async_copy(
                        dstp.at[pl.ds(off, B)], didx, isems[1]).wait()
                    for i in range(B // L):
                        sidxh[pl.ds(i * L, L)] = sidx[pl.ds(i * L, L)] + goff
                    pltpu.async_copy(as_t.at[sidx], asv, gs[0])
                    pltpu.async_copy(ad_t.at[didx], adv, gs[1])
                    pltpu.async_copy(h_ts.at[sidxh], hrows, gs[2])

            prime(0, 0)
            prime(1, 1)
            plsc.subcore_barrier()

            def pair(k2, _):
                compute(0, k2 >= 1, k2 < NB2 // 2 - 1, 2 * k2 + 2)
                compute(1, k2 >= 1, k2 < NB2 // 2 - 1, 2 * k2 + 3)
                return 0
            lax.fori_loop(0, NB2 // 2, pair, 0)
            for x in range(2):
                _, _, didxs_x, _, _, _, wv_x, _, msgv_x, _, _, dsem_x, \
                    asem_x = bufs[x]
                pltpu.make_async_copy(wv_x, den_sh.at[didxs_x], dsem_x).wait()
                pltpu.make_async_copy(
                    msgv_x, acc_sh.at[didxs_x], asem_x).wait()
            plsc.subcore_barrier()

            def drain(k, _):
                c = sid * ZC + k

                @pl.when(c < N // ZR)
                def _():
                    pltpu.sync_copy(acc_sh.at[pl.ds(c * ZR, ZR)],
                                    acc_out.at[pl.ds(goff + c * ZR, ZR)])

                    @pl.when(gl == 1)
                    def _():
                        pltpu.sync_copy(
                            den_sh.at[pl.ds(c * ZR, ZR)],
                            den_out.at[pl.ds(cid * N + c * ZR, ZR)])
                return 0
            lax.fori_loop(0, ZC, drain, 0)
            plsc.subcore_barrier()
            return 0
        lax.fori_loop(0, 2, one_group, 0)

    return body


_sc_pass2m = _make_sc_pass2m()

BN = 400


def _tc_prep1(x, W1, asf1, adf1):
    def body(x_ref, w_ref, as_ref, ad_ref, h_out, aso, ado):
        h = jnp.dot(x_ref[...], w_ref[...], preferred_element_type=jnp.float32, precision=lax.Precision.HIGHEST)
        h_out[...] = h
        c = lax.broadcasted_iota(jnp.int32, (128, 8), 0)
        hh = lax.broadcasted_iota(jnp.int32, (128, 8), 1)
        S = (c // 16 == hh).astype(jnp.float32)
        aso[...] = jnp.dot(h * as_ref[...], S, preferred_element_type=jnp.float32, precision=lax.Precision.HIGHEST)
        ado[...] = jnp.dot(h * ad_ref[...], S, preferred_element_type=jnp.float32, precision=lax.Precision.HIGHEST)

    return pl.pallas_call(
        body,
        grid=(N // BN,),
        in_specs=[
            pl.BlockSpec((BN, 128), lambda i: (i, 0)),
            pl.BlockSpec((128, 128), lambda i: (0, 0)),
            pl.BlockSpec((1, 128), lambda i: (0, 0)),
            pl.BlockSpec((1, 128), lambda i: (0, 0)),
        ],
        out_specs=(
            pl.BlockSpec((BN, 128), lambda i: (i, 0)),
            pl.BlockSpec((BN, 8), lambda i: (i, 0)),
            pl.BlockSpec((BN, 8), lambda i: (i, 0)),
        ),
        out_shape=(
            jax.ShapeDtypeStruct((N, 128), jnp.float32),
            jax.ShapeDtypeStruct((N, 8), jnp.float32),
            jax.ShapeDtypeStruct((N, 8), jnp.float32),
        ),
    )(x, W1, asf1, adf1)


def _tc_mid(acc0, acc1, den0, den1, b1r, W2, asf2, adf2):
    def body(a0, a1, d0, d1, b1_ref, w2, as_ref, ad_ref, h2o, aso, ado):
        den = d0[...] + d1[...] + 1e-16
        hh = lax.broadcasted_iota(jnp.int32, (8, 128), 0)
        cc = lax.broadcasted_iota(jnp.int32, (8, 128), 1)
        R8 = (hh == cc // 16).astype(jnp.float32)
        denrep = jnp.dot(den, R8, preferred_element_type=jnp.float32, precision=lax.Precision.HIGHEST)
        v = (a0[...] + a1[...]) / denrep + b1_ref[...]
        x1 = jnp.where(v > 0, v, jnp.exp(v) - 1.0)
        h2 = jnp.dot(x1, w2[...], preferred_element_type=jnp.float32, precision=lax.Precision.HIGHEST)
        h2o[...] = h2
        c2 = lax.broadcasted_iota(jnp.int32, (512, 8), 0)
        h2i = lax.broadcasted_iota(jnp.int32, (512, 8), 1)
        S2 = (c2 // 64 == h2i).astype(jnp.float32)
        aso[...] = jnp.dot(h2 * as_ref[...], S2, preferred_element_type=jnp.float32, precision=lax.Precision.HIGHEST)
        ado[...] = jnp.dot(h2 * ad_ref[...], S2, preferred_element_type=jnp.float32, precision=lax.Precision.HIGHEST)

    return pl.pallas_call(
        body,
        grid=(N // BN,),
        in_specs=[
            pl.BlockSpec((BN, 128), lambda i: (i, 0)),
            pl.BlockSpec((BN, 128), lambda i: (i, 0)),
            pl.BlockSpec((BN, 8), lambda i: (i, 0)),
            pl.BlockSpec((BN, 8), lambda i: (i, 0)),
            pl.BlockSpec((1, 128), lambda i: (0, 0)),
            pl.BlockSpec((128, 512), lambda i: (0, 0)),
            pl.BlockSpec((1, 512), lambda i: (0, 0)),
            pl.BlockSpec((1, 512), lambda i: (0, 0)),
        ],
        out_specs=(
            pl.BlockSpec((BN, 512), lambda i: (i, 0)),
            pl.BlockSpec((BN, 8), lambda i: (i, 0)),
            pl.BlockSpec((BN, 8), lambda i: (i, 0)),
        ),
        out_shape=(
            jax.ShapeDtypeStruct((N, 512), jnp.float32),
            jax.ShapeDtypeStruct((N, 8), jnp.float32),
            jax.ShapeDtypeStruct((N, 8), jnp.float32),
        ),
    )(acc0, acc1, den0, den1, b1r, W2, asf2, adf2)


def _tc_final(accs, dens, b2r):
    def body(*refs):
        a = refs[0:4]
        d = refs[4:8]
        b2_ref = refs[8]
        out = refs[9]
        hh = lax.broadcasted_iota(jnp.int32, (2, 128), 0)
        cc = lax.broadcasted_iota(jnp.int32, (2, 128), 1)
        R2 = (hh == cc // 64).astype(jnp.float32)
        cf = lax.broadcasted_iota(jnp.int32, (128, 64), 0)
        of = lax.broadcasted_iota(jnp.int32, (128, 64), 1)
        F = (cf % 64 == of).astype(jnp.float32)
        tot = jnp.zeros((BN, 64), jnp.float32)
        for g in range(4):
            den = d[g][...] * 0.5 + 1e-16
            denrep = jnp.dot(den, R2, preferred_element_type=jnp.float32, precision=lax.Precision.HIGHEST)
            v = a[g][...] / denrep
            tot = tot + jnp.dot(v, F, preferred_element_type=jnp.float32, precision=lax.Precision.HIGHEST)
        out[...] = tot * (1.0 / 8.0) + b2_ref[...]

    return pl.pallas_call(
        body,
        grid=(N // BN,),
        in_specs=(
            [pl.BlockSpec((BN, 128), lambda i: (i, 0)) for _ in range(4)]
            + [pl.BlockSpec((BN, 2), lambda i: (i, 0)) for _ in range(4)]
            + [pl.BlockSpec((1, 64), lambda i: (0, 0))]
        ),
        out_specs=pl.BlockSpec((BN, 64), lambda i: (i, 0)),
        out_shape=jax.ShapeDtypeStruct((N, 64), jnp.float32),
    )(*accs, *dens, b2r)


def kernel(x, edge_index, W1, a_src1, a_dst1, b1, W2, a_src2, a_dst2, b2):
    ei = edge_index.astype(jnp.int32)
    srcp = ei[0]
    dstp = ei[1]

    h1, as1, ad1 = _tc_prep1(x, W1, a_src1.reshape(1, -1), a_dst1.reshape(1, -1))
    accp, denp = _sc_pass1(srcp, dstp, as1, ad1, h1)
    h2, as2, ad2 = _tc_mid(accp[:N], accp[N:], denp[:N], denp[N:],
                           b1.reshape(1, -1), W2,
                           a_src2.reshape(1, -1), a_dst2.reshape(1, -1))
    h2s = h2.reshape(N, 4, 128).transpose(1, 0, 2).reshape(4 * N, 128)
    perm = []
    for c32 in range(0, 128, 32):
        for i in range(16):
            perm += [c32 + i, c32 + 16 + i]
    h2s = h2s.astype(jnp.bfloat16)[:, jnp.asarray(perm, jnp.int32)]
    acc4, den2o = _sc_pass2m(srcp, dstp, as2, ad2, h2s)
    accs = [acc4[g * N:(g + 1) * N] for g in range(4)]
    dens = [den2o[:N, 2 * g:2 * g + 2] for g in range(4)]
    return _tc_final(accs, dens, b2.reshape(1, -1))


# speedup vs baseline: 1.2201x; 1.0578x over previous
import functools

import jax
import jax.numpy as jnp
from jax import lax
from jax.experimental import pallas as pl
from jax.experimental.pallas import tpu as pltpu
from jax.experimental.pallas import tpu_sc as plsc

N = 10000
E = 320000
D = 128
NC, NS, L = 2, 16, 16
NW = NC * NS
EPT = E // NW
B = 80
NB = EPT // B
ZR = 80
ZC = 8

_mesh = plsc.VectorSubcoreMesh(
    core_axis_name="c", subcore_axis_name="s", num_cores=NC, num_subcores=NS)


H = 8


def _make_sc_pass(CH, hoff):
    HB = B * H // L
    out_type = (
        jax.ShapeDtypeStruct((NC * N, D), jnp.float32),
        jax.ShapeDtypeStruct((NC * N, H), jnp.float32),
    )
    scratch_types = [
        pltpu.VMEM((2, B), jnp.int32),
        pltpu.VMEM((2, B), jnp.int32),
        pltpu.VMEM((2, B), jnp.int32),
        pltpu.VMEM((2, B, H), jnp.float32),
        pltpu.VMEM((2, B, H), jnp.float32),
        pltpu.VMEM((2, B, H), jnp.float32),
        pltpu.VMEM((2, B, D), jnp.float32),
        pltpu.VMEM((2, B, D), jnp.float32),
        pltpu.VMEM_SHARED((N, D), jnp.float32),
        pltpu.VMEM_SHARED((N, H), jnp.float32),
    ] + [pltpu.SemaphoreType.DMA] * 14

    @functools.partial(
        pl.kernel, out_type=out_type, mesh=_mesh,
        scratch_types=scratch_types,
        compiler_params=pltpu.CompilerParams(
            needs_layout_passes=False, use_tc_tiling_on_sc=False))
    def body(srcp, dstp, as_t, ad_t, h_t, acc_out, den_out,
             sidx2, didx2, didxs2, asv2, adv2, wv2, hrows2, msgv2,
             acc_sh, den_sh, *sems):
        cid = lax.axis_index("c")
        sid = lax.axis_index("s")
        zv = jnp.zeros((L,), jnp.float32)
        iot = lax.iota(jnp.int32, L)
        bufs = [
            (sidx2.at[i], didx2.at[i], didxs2.at[i], asv2.at[i], adv2.at[i],
             wv2.at[i], hrows2.at[i], msgv2.at[i],
             sems[7 * i:7 * i + 3],
             sems[7 * i + 3:7 * i + 5],
             sems[7 * i + 5],
             sems[7 * i + 6])
            for i in range(2)
        ]
        hrows_z = hrows2.at[0]
        wv_z = wv2.at[0]

        def zh(i, _):
            hrows_z[i // (D // L), pl.ds((i % (D // L)) * L, L)] = zv
            return 0
        lax.fori_loop(0, B * (D // L), zh, 0)

        def zw(i, _):
            p = i * L + iot
            plsc.store_scatter(wv_z, [p // H, p % H], zv)
            return 0
        lax.fori_loop(0, HB, zw, 0)

        def zcp(k, _):
            c = sid * ZC + k

            @pl.when(c < N // ZR)
            def _():
                pltpu.sync_copy(hrows_z.at[pl.ds(0, ZR)],
                                acc_sh.at[pl.ds(c * ZR, ZR)])
                pltpu.sync_copy(wv_z.at[pl.ds(0, ZR)],
                                den_sh.at[pl.ds(c * ZR, ZR)])
            return 0
        lax.fori_loop(0, ZC, zcp, 0)

        ebase = (cid * NS + sid) * EPT

        def prime(x, blk):
            sidx, didx, _, asv, adv, _, hrows, _, gs, _, _, _ = bufs[x]
            off = ebase + blk * B
            pltpu.sync_copy(srcp.at[pl.ds(off, B)], sidx)
            pltpu.sync_copy(dstp.at[pl.ds(off, B)], didx)
            pltpu.async_copy(as_t.at[sidx], asv, gs[0])
            pltpu.async_copy(ad_t.at[didx], adv, gs[1])
            pltpu.async_copy(h_t.at[sidx], hrows, gs[2])

        def compute(x, steady, pf, pf_blk):
            sidx, didx, didxs, asv, adv, wv, hrows, msgv, gs, isems, dsem, \
                asem = bufs[x]
            pltpu.make_async_copy(as_t.at[sidx], asv, gs[0]).wait()
            pltpu.make_async_copy(ad_t.at[didx], adv, gs[1]).wait()
            pltpu.make_async_copy(h_t.at[sidx], hrows, gs[2]).wait()

            @pl.when(steady)
            def _():
                pltpu.make_async_copy(wv, den_sh.at[didxs], dsem).wait()
                pltpu.make_async_copy(msgv, acc_sh.at[didxs], asem).wait()

            @plsc.parallel_loop(0, HB, 1, unroll=4)
            def wb(v):
                p = v * L + iot
                rows = p // H
                cols = p % H
                e = (plsc.load_gather(asv, [rows, cols])
                     + plsc.load_gather(adv, [rows, cols]))
                e = jnp.where(e >= 0, e, 0.2 * e)
                w = jnp.exp(e)
                plsc.store_scatter(wv, [rows, cols], w)

            for i in range(B // L):
                didxs[pl.ds(i * L, L)] = didx[pl.ds(i * L, L)]
            off = ebase + pf_blk * B
            if pf is None:
                pltpu.sync_copy(wv, den_sh.at[didxs], add=True)
            else:
                pltpu.async_copy(wv, den_sh.at[didxs], dsem, add=True)

                @pl.when(pf)
                def _():
                    pltpu.async_copy(srcp.at[pl.ds(off, B)], sidx, isems[0])
                    pltpu.async_copy(dstp.at[pl.ds(off, B)], didx, isems[1])

            @plsc.parallel_loop(0, B, 1, unroll=4)
            def mb(b):
                rowspl = jnp.broadcast_to(b, (L,))
                for hl in range(D // CH):
                    wvec = plsc.load_gather(
                        wv, [rowspl, jnp.full((L,), hoff + hl, jnp.int32)])
                    for j in range(CH // L):
                        c0 = hl * CH + j * L
                        msgv[b, pl.ds(c0, L)] = hrows[b, pl.ds(c0, L)] * wvec

            if pf is None:
                pltpu.sync_copy(msgv, acc_sh.at[didxs], add=True)
            else:
                pltpu.async_copy(msgv, acc_sh.at[didxs], asem, add=True)

                @pl.when(pf)
                def _():
                    pltpu.make_async_copy(
                        srcp.at[pl.ds(off, B)], sidx, isems[0]).wait()
                    pltpu.make_async_copy(
                        dstp.at[pl.ds(off, B)], didx, isems[1]).wait()
                    pltpu.async_copy(as_t.at[sidx], asv, gs[0])
                    pltpu.async_copy(ad_t.at[didx], adv, gs[1])
                    pltpu.async_copy(h_t.at[sidx], hrows, gs[2])

        prime(0, 0)
        prime(1, 1)
        plsc.subcore_barrier()
        true_ = jnp.bool_(True)

        def pair(k2, _):
            compute(0, k2 >= 1, true_, 2 * k2 + 2)
            compute(1, k2 >= 1, k2 < NB // 2 - 1, 2 * k2 + 3)
            return 0
        lax.fori_loop(0, NB // 2, pair, 0)
        compute(0, true_, None, 0)
        _, _, didxs_b, _, _, wv_b, _, msgv_b, _, _, dsem_b, asem_b = bufs[1]
        pltpu.make_async_copy(wv_b, den_sh.at[didxs_b], dsem_b).wait()
        pltpu.make_async_copy(msgv_b, acc_sh.at[didxs_b], asem_b).wait()
        plsc.subcore_barrier()

        def drain(k, _):
            c = sid * ZC + k

            @pl.when(c < N // ZR)
            def _():
                pltpu.sync_copy(acc_sh.at[pl.ds(c * ZR, ZR)],
                                acc_out.at[pl.ds(cid * N + c * ZR, ZR)])
                pltpu.sync_copy(den_sh.at[pl.ds(c * ZR, ZR)],
                                den_out.at[pl.ds(cid * N + c * ZR, ZR)])
            return 0
        lax.fori_loop(0, ZC, drain, 0)

    return body


_sc_pass1 = _make_sc_pass(16, 0)

EPT2 = E // NS
NB2 = EPT2 // B


def _make_sc_pass2m():
    CH = 64
    HB = B * H // L
    out_type = (
        jax.ShapeDtypeStruct((4 * N, D), jnp.float32),
        jax.ShapeDtypeStruct((NC * N, H), jnp.float32),
    )
    scratch_types = [
        pltpu.VMEM((2, B), jnp.int32),
        pltpu.VMEM((2, B), jnp.int32),
        pltpu.VMEM((2, B), jnp.int32),
        pltpu.VMEM((2, B), jnp.int32),
        pltpu.VMEM((2, B, H), jnp.float32),
        pltpu.VMEM((2, B, H), jnp.float32),
        pltpu.VMEM((2, B, H), jnp.float32),
        pltpu.VMEM((2, B, D), jnp.bfloat16),
        pltpu.VMEM((2, B, D), jnp.float32),
        pltpu.VMEM_SHARED((N, D), jnp.float32),
        pltpu.VMEM_SHARED((N, H), jnp.float32),
    ] + [pltpu.SemaphoreType.DMA] * 14

    @functools.partial(
        pl.kernel, out_type=out_type, mesh=_mesh,
        scratch_types=scratch_types,
        compiler_params=pltpu.CompilerParams(
            needs_layout_passes=False, use_tc_tiling_on_sc=False))
    def body(srcp, dstp, as_t, ad_t, h_ts, acc_out, den_out,
             sidx2, didx2, didxs2, sidxh2, asv2, adv2, wv2, hrows2, msgv2,
             acc_sh, den_sh, *sems):
        cid = lax.axis_index("c")
        sid = lax.axis_index("s")
        zv = jnp.zeros((L,), jnp.float32)
        iot = lax.iota(jnp.int32, L)
        bufs = [
            (sidx2.at[i], didx2.at[i], didxs2.at[i], sidxh2.at[i],
             asv2.at[i], adv2.at[i], wv2.at[i], hrows2.at[i], msgv2.at[i],
             sems[7 * i:7 * i + 3], sems[7 * i + 3:7 * i + 5],
             sems[7 * i + 5], sems[7 * i + 6])
            for i in range(2)
        ]
        msgv_z = msgv2.at[0]
        wv_z = wv2.at[0]
        ebase = sid * EPT2

        def one_group(gl, _):
            g = 2 * cid + gl
            goff = g * N
            hoff = 2 * g

            def zh(i, _):
                msgv_z[i // (D // L), pl.ds((i % (D // L)) * L, L)] = zv
                return 0
            lax.fori_loop(0, B * (D // L), zh, 0)

            def zw(i, _):
                p = i * L + iot
                plsc.store_scatter(wv_z, [p // H, p % H], zv)
                return 0
            lax.fori_loop(0, HB, zw, 0)

            def zcp(k, _):
                c = sid * ZC + k

                @pl.when(c < N // ZR)
                def _():
                    pltpu.sync_copy(msgv_z.at[pl.ds(0, ZR)],
                                    acc_sh.at[pl.ds(c * ZR, ZR)])

                    @pl.when(gl == 0)
                    def _():
                        pltpu.sync_copy(wv_z.at[pl.ds(0, ZR)],
                                        den_sh.at[pl.ds(c * ZR, ZR)])
                return 0
            lax.fori_loop(0, ZC, zcp, 0)

            def prime(x, blk):
                sidx, didx, _, sidxh, asv, adv, _, hrows, _, gs, _, _, _ = \
                    bufs[x]
                off = ebase + blk * B
                pltpu.sync_copy(srcp.at[pl.ds(off, B)], sidx)
                pltpu.sync_copy(dstp.at[pl.ds(off, B)], didx)
                for i in range(B // L):
                    sidxh[pl.ds(i * L, L)] = sidx[pl.ds(i * L, L)] * 4 + g
                pltpu.async_copy(as_t.at[sidx], asv, gs[0])
                pltpu.async_copy(ad_t.at[didx], adv, gs[1])
                pltpu.async_copy(h_ts.at[sidxh], hrows, gs[2])

            def compute(x, steady, pf, pf_blk):
                sidx, didx, didxs, sidxh, asv, adv, wv, hrows, msgv, gs, \
                    isems, dsem, asem = bufs[x]
                pltpu.make_async_copy(as_t.at[sidx], asv, gs[0]).wait()
                pltpu.make_async_copy(ad_t.at[didx], adv, gs[1]).wait()
                pltpu.make_async_copy(h_ts.at[sidxh], hrows, gs[2]).wait()

                @pl.when(steady)
                def _():
                    pltpu.make_async_copy(wv, den_sh.at[didxs], dsem).wait()
                    pltpu.make_async_copy(msgv, acc_sh.at[didxs], asem).wait()

                @plsc.parallel_loop(0, HB, 1, unroll=4)
                def wb(v):
                    p = v * L + iot
                    rows = p // H
                    cols = p % H
                    e = (plsc.load_gather(asv, [rows, cols])
                         + plsc.load_gather(adv, [rows, cols]))
                    e = jnp.where(e >= 0, e, 0.2 * e)
                    w = jnp.exp(e)
                    plsc.store_scatter(wv, [rows, cols], w)

                for i in range(B // L):
                    didxs[pl.ds(i * L, L)] = didx[pl.ds(i * L, L)]
                off = ebase + pf_blk * B
                pltpu.async_copy(wv, den_sh.at[didxs], dsem, add=True)

                @pl.when(pf)
                def _():
                    pltpu.async_copy(srcp.at[pl.ds(off, B)], sidx, isems[0])
                    pltpu.async_copy(dstp.at[pl.ds(off, B)], didx, isems[1])

                @plsc.parallel_loop(0, B, 1, unroll=4)
                def mb(b):
                    rowspl = jnp.broadcast_to(b, (L,))
                    for hl in range(D // CH):
                        wvec = plsc.load_gather(
                            wv, [rowspl, jnp.broadcast_to(hoff + hl, (L,))])
                        for j2 in range(CH // (2 * L)):
                            c32 = hl * CH + j2 * 2 * L
                            v = hrows[b, pl.ds(c32, 2 * L)]
                            lo, hi = plsc.unpack(
                                v, format=plsc.PackFormat.INTERLEAVED)
                            msgv[b, pl.ds(c32, L)] = lo * wvec
                            msgv[b, pl.ds(c32 + L, L)] = hi * wvec

                pltpu.async_copy(msgv, acc_sh.at[didxs], asem, add=True)

                @pl.when(pf)
                def _():
                    pltpu.make_async_copy(
                        srcp.at[pl.ds(off, B)], sidx, isems[0]).wait()
                    pltpu.make_async_copy(
                        dstp.at[pl.ds(off, B)], didx, isems[1]).wait()
                    for i in range(B // L):
                        sidxh[pl.ds(i * L, L)] = sidx[pl.ds(i * L, L)] * 4 + g
                    pltpu.async_copy(as_t.at[sidx], asv, gs[0])
                    pltpu.async_copy(ad_t.at[didx], adv, gs[1])
                    pltpu.async_copy(h_ts.at[sidxh], hrows, gs[2])

            prime(0, 0)
            prime(1, 1)
            plsc.subcore_barrier()

            def pair(k2, _):
                compute(0, k2 >= 1, k2 < NB2 // 2 - 1, 2 * k2 + 2)
                compute(1, k2 >= 1, k2 < NB2 // 2 - 1, 2 * k2 + 3)
                return 0
            lax.fori_loop(0, NB2 // 2, pair, 0)
            for x in range(2):
                _, _, didxs_x, _, _, _, wv_x, _, msgv_x, _, _, dsem_x, \
                    asem_x = bufs[x]
                pltpu.make_async_copy(wv_x, den_sh.at[didxs_x], dsem_x).wait()
                pltpu.make_async_copy(
                    msgv_x, acc_sh.at[didxs_x], asem_x).wait()
            plsc.subcore_barrier()

            def drain(k, _):
                c = sid * ZC + k

                @pl.when(c < N // ZR)
                def _():
                    pltpu.sync_copy(acc_sh.at[pl.ds(c * ZR, ZR)],
                                    acc_out.at[pl.ds(goff + c * ZR, ZR)])

                    @pl.when(gl == 1)
                    def _():
                        pltpu.sync_copy(
                            den_sh.at[pl.ds(c * ZR, ZR)],
                            den_out.at[pl.ds(cid * N + c * ZR, ZR)])
                return 0
            lax.fori_loop(0, ZC, drain, 0)
            plsc.subcore_barrier()
            return 0
        lax.fori_loop(0, 2, one_group, 0)

    return body


_sc_pass2m = _make_sc_pass2m()

BN = 400


def _tc_prep1(x, W1, asf1, adf1):
    def body(x_ref, w_ref, as_ref, ad_ref, h_out, aso, ado):
        h = jnp.dot(x_ref[...], w_ref[...], preferred_element_type=jnp.float32, precision=lax.Precision.HIGHEST)
        h_out[...] = h
        c = lax.broadcasted_iota(jnp.int32, (128, 8), 0)
        hh = lax.broadcasted_iota(jnp.int32, (128, 8), 1)
        S = (c // 16 == hh).astype(jnp.float32)
        aso[...] = jnp.dot(h * as_ref[...], S, preferred_element_type=jnp.float32, precision=lax.Precision.HIGHEST)
        ado[...] = jnp.dot(h * ad_ref[...], S, preferred_element_type=jnp.float32, precision=lax.Precision.HIGHEST)

    return pl.pallas_call(
        body,
        grid=(N // BN,),
        in_specs=[
            pl.BlockSpec((BN, 128), lambda i: (i, 0)),
            pl.BlockSpec((128, 128), lambda i: (0, 0)),
            pl.BlockSpec((1, 128), lambda i: (0, 0)),
            pl.BlockSpec((1, 128), lambda i: (0, 0)),
        ],
        out_specs=(
            pl.BlockSpec((BN, 128), lambda i: (i, 0)),
            pl.BlockSpec((BN, 8), lambda i: (i, 0)),
            pl.BlockSpec((BN, 8), lambda i: (i, 0)),
        ),
        out_shape=(
            jax.ShapeDtypeStruct((N, 128), jnp.float32),
            jax.ShapeDtypeStruct((N, 8), jnp.float32),
            jax.ShapeDtypeStruct((N, 8), jnp.float32),
        ),
    )(x, W1, asf1, adf1)


def _tc_mid(acc0, acc1, den0, den1, b1r, W2, asf2, adf2):
    def body(a0, a1, d0, d1, b1_ref, w2, as_ref, ad_ref, h2o, aso, ado):
        den = d0[...] + d1[...] + 1e-16
        hh = lax.broadcasted_iota(jnp.int32, (8, 128), 0)
        cc = lax.broadcasted_iota(jnp.int32, (8, 128), 1)
        R8 = (hh == cc // 16).astype(jnp.float32)
        denrep = jnp.dot(den, R8, preferred_element_type=jnp.float32, precision=lax.Precision.HIGHEST)
        v = (a0[...] + a1[...]) / denrep + b1_ref[...]
        x1 = jnp.where(v > 0, v, jnp.exp(v) - 1.0)
        h2 = jnp.dot(x1, w2[...], preferred_element_type=jnp.float32, precision=lax.Precision.HIGHEST)
        c2 = lax.broadcasted_iota(jnp.int32, (512, 8), 0)
        h2i = lax.broadcasted_iota(jnp.int32, (512, 8), 1)
        S2 = (c2 // 64 == h2i).astype(jnp.float32)
        aso[...] = jnp.dot(h2 * as_ref[...], S2, preferred_element_type=jnp.float32, precision=lax.Precision.HIGHEST)
        ado[...] = jnp.dot(h2 * ad_ref[...], S2, preferred_element_type=jnp.float32, precision=lax.Precision.HIGHEST)
        jj = lax.broadcasted_iota(jnp.int32, (512, 512), 1)
        local = jj % 128
        c32 = (local // 32) * 32
        pos = local % 32
        lp = jnp.where(pos % 2 == 0, c32 + pos // 2, c32 + 16 + (pos - 1) // 2)
        pm = (jj - local) + lp
        aa = lax.broadcasted_iota(jnp.int32, (512, 512), 0)
        P = (aa == pm).astype(jnp.bfloat16)
        h2o[...] = jnp.dot(h2.astype(jnp.bfloat16), P,
                           preferred_element_type=jnp.float32
                           ).astype(jnp.bfloat16)

    return pl.pallas_call(
        body,
        grid=(N // BN,),
        in_specs=[
            pl.BlockSpec((BN, 128), lambda i: (i, 0)),
            pl.BlockSpec((BN, 128), lambda i: (i, 0)),
            pl.BlockSpec((BN, 8), lambda i: (i, 0)),
            pl.BlockSpec((BN, 8), lambda i: (i, 0)),
            pl.BlockSpec((1, 128), lambda i: (0, 0)),
            pl.BlockSpec((128, 512), lambda i: (0, 0)),
            pl.BlockSpec((1, 512), lambda i: (0, 0)),
            pl.BlockSpec((1, 512), lambda i: (0, 0)),
        ],
        out_specs=(
            pl.BlockSpec((BN, 512), lambda i: (i, 0)),
            pl.BlockSpec((BN, 8), lambda i: (i, 0)),
            pl.BlockSpec((BN, 8), lambda i: (i, 0)),
        ),
        out_shape=(
            jax.ShapeDtypeStruct((N, 512), jnp.bfloat16),
            jax.ShapeDtypeStruct((N, 8), jnp.float32),
            jax.ShapeDtypeStruct((N, 8), jnp.float32),
        ),
    )(acc0, acc1, den0, den1, b1r, W2, asf2, adf2)


def _tc_final(accs, dens, b2r):
    def body(*refs):
        a = refs[0:4]
        d = refs[4:8]
        b2_ref = refs[8]
        out = refs[9]
        hh = lax.broadcasted_iota(jnp.int32, (2, 128), 0)
        cc = lax.broadcasted_iota(jnp.int32, (2, 128), 1)
        R2 = (hh == cc // 64).astype(jnp.float32)
        cf = lax.broadcasted_iota(jnp.int32, (128, 64), 0)
        of = lax.broadcasted_iota(jnp.int32, (128, 64), 1)
        F = (cf % 64 == of).astype(jnp.float32)
        tot = jnp.zeros((BN, 64), jnp.float32)
        for g in range(4):
            den = d[g][...] * 0.5 + 1e-16
            denrep = jnp.dot(den, R2, preferred_element_type=jnp.float32, precision=lax.Precision.HIGHEST)
            v = a[g][...] / denrep
            tot = tot + jnp.dot(v, F, preferred_element_type=jnp.float32, precision=lax.Precision.HIGHEST)
        out[...] = tot * (1.0 / 8.0) + b2_ref[...]

    return pl.pallas_call(
        body,
        grid=(N // BN,),
        in_specs=(
            [pl.BlockSpec((BN, 128), lambda i: (i, 0)) for _ in range(4)]
            + [pl.BlockSpec((BN, 2), lambda i: (i, 0)) for _ in range(4)]
            + [pl.BlockSpec((1, 64), lambda i: (0, 0))]
        ),
        out_specs=pl.BlockSpec((BN, 64), lambda i: (i, 0)),
        out_shape=jax.ShapeDtypeStruct((N, 64), jnp.float32),
    )(*accs, *dens, b2r)


def kernel(x, edge_index, W1, a_src1, a_dst1, b1, W2, a_src2, a_dst2, b2):
    ei = edge_index.astype(jnp.int32)
    srcp = ei[0]
    dstp = ei[1]

    h1, as1, ad1 = _tc_prep1(x, W1, a_src1.reshape(1, -1), a_dst1.reshape(1, -1))
    accp, denp = _sc_pass1(srcp, dstp, as1, ad1, h1)
    h2, as2, ad2 = _tc_mid(accp[:N], accp[N:], denp[:N], denp[N:],
                           b1.reshape(1, -1), W2,
                           a_src2.reshape(1, -1), a_dst2.reshape(1, -1))
    h2s = h2.reshape(4 * N, 128)
    acc4, den2o = _sc_pass2m(srcp, dstp, as2, ad2, h2s)
    accs = [acc4[g * N:(g + 1) * N] for g in range(4)]
    dens = [den2o[:N, 2 * g:2 * g + 2] for g in range(4)]
    return _tc_final(accs, dens, b2.reshape(1, -1))
